# baseline (speedup 1.0000x reference)
import functools

import jax
import jax.numpy as jnp
from jax import lax
from jax.experimental import pallas as pl
from jax.experimental.pallas import tpu as pltpu
from jax.experimental.pallas import tpu_sc as plsc

_BATCH = 16384
_NC, _NS = 2, 16
_NW = _NC * _NS
_B_PER_W = _BATCH // _NW


def _make_gather():
    mesh = plsc.VectorSubcoreMesh(core_axis_name="c", subcore_axis_name="s")

    half = _B_PER_W // 2

    @functools.partial(
        pl.kernel,
        mesh=mesh,
        out_type=jax.ShapeDtypeStruct((_BATCH,), jnp.float32),
        scratch_types=[
            pltpu.VMEM((_B_PER_W,), jnp.int32),
            pltpu.VMEM((_B_PER_W,), jnp.float32),
            pltpu.SemaphoreType.DMA,
            pltpu.SemaphoreType.DMA,
            pltpu.SemaphoreType.DMA,
        ],
    )
    def gather_kernel(v_hbm, idx_hbm, out_hbm, idx_v, out_v, isem, gsem, osem):
        wid = lax.axis_index("s") * _NC + lax.axis_index("c")
        base = wid * _B_PER_W
        idx_cp = [
            pltpu.async_copy(
                idx_hbm.at[pl.ds(base + h * half, half)],
                idx_v.at[pl.ds(h * half, half)],
                isem,
            )
            for h in range(2)
        ]
        g_cp = []
        for h in range(2):
            idx_cp[h].wait()
            g_cp.append(
                pltpu.async_copy(
                    v_hbm.at[idx_v.at[pl.ds(h * half, half)]],
                    out_v.at[pl.ds(h * half, half)],
                    gsem,
                )
            )
        o_cp = []
        for h in range(2):
            g_cp[h].wait()
            o_cp.append(
                pltpu.async_copy(
                    out_v.at[pl.ds(h * half, half)],
                    out_hbm.at[pl.ds(base + h * half, half)],
                    osem,
                )
            )
        for o in o_cp:
            o.wait()

    return gather_kernel


_GATHER = _make_gather()


def kernel(v, indices):
    return _GATHER(v, indices)


# speedup vs baseline: 1.0043x; 1.0043x over previous
import functools

import jax
import jax.numpy as jnp
from jax import lax
from jax.experimental import pallas as pl
from jax.experimental.pallas import tpu as pltpu
from jax.experimental.pallas import tpu_sc as plsc

_BATCH = 16384
_NC, _NS = 2, 16
_NW = _NC * _NS
_B_PER_W = _BATCH // _NW


def _make_gather():
    mesh = plsc.VectorSubcoreMesh(core_axis_name="c", subcore_axis_name="s")

    @functools.partial(
        pl.kernel,
        mesh=mesh,
        out_type=jax.ShapeDtypeStruct((_BATCH,), jnp.float32),
        scratch_types=[
            pltpu.VMEM((_B_PER_W,), jnp.int32),
            pltpu.VMEM((_B_PER_W,), jnp.float32),
            pltpu.SemaphoreType.DMA,
        ],
    )
    def gather_kernel(v_hbm, idx_hbm, out_hbm, idx_v, out_v, sem):
        wid = lax.axis_index("s") * _NC + lax.axis_index("c")
        base = wid * _B_PER_W
        pltpu.sync_copy(idx_hbm.at[pl.ds(base, _B_PER_W)], idx_v)
        pltpu.async_copy(v_hbm.at[idx_v], out_v, sem).wait()
        pltpu.sync_copy(out_v, out_hbm.at[pl.ds(base, _B_PER_W)])

    return gather_kernel


_GATHER = _make_gather()


def kernel(v, indices):
    return _GATHER(v, indices)
